# Initial kernel scaffold; baseline (speedup 1.0000x reference)
import jax
import jax.numpy as jnp
from jax.experimental import pallas as pl


def kernel(stims, embed, atn_idx, lens):
    raise NotImplementedError("write your pallas kernel here")


# speedup vs baseline: 1.8007x; 1.8007x over previous
import functools

import jax
import jax.numpy as jnp
from jax import lax
from jax.experimental import pallas as pl
from jax.experimental.pallas import tpu as pltpu
from jax.experimental.pallas import tpu_sc as plsc

B, J, K, H, V = 16, 4, 2048, 256, 65536
PAIRS = B * J
L = 16
NC, NS = 2, 16
NW = NC * NS
PPW = PAIRS // NW
VBLK = 4096


def _tc_scores(stims, embed):
    def mm(stims_ref, emb_ref, out_ref):
        out_ref[...] = lax.dot_general(
            stims_ref[...], emb_ref[...],
            dimension_numbers=(((1,), (1,)), ((), ())),
            preferred_element_type=jnp.float32,
            precision=lax.Precision.HIGHEST,
        )

    return pl.pallas_call(
        mm,
        grid=(V // VBLK,),
        in_specs=[
            pl.BlockSpec((B, H), lambda i: (0, 0)),
            pl.BlockSpec((VBLK, H), lambda i: (i, 0)),
        ],
        out_specs=pl.BlockSpec((B, VBLK), lambda i: (0, i)),
        out_shape=jax.ShapeDtypeStruct((B, V), jnp.float32),
    )(stims, embed)


def _sc_gather_argmax(scores, idx, lens):
    mesh = plsc.VectorSubcoreMesh(core_axis_name="c", subcore_axis_name="s")

    @functools.partial(
        pl.kernel,
        mesh=mesh,
        compiler_params=pltpu.CompilerParams(needs_layout_passes=False),
        out_type=[
            jax.ShapeDtypeStruct((PAIRS, K), jnp.float32),
            jax.ShapeDtypeStruct((PAIRS, L), jnp.int32),
        ],
        scratch_types=[
            pltpu.VMEM((V,), jnp.float32),
            pltpu.VMEM((K,), jnp.int32),
            pltpu.VMEM((K,), jnp.float32),
            pltpu.VMEM((L,), jnp.int32),
            pltpu.VMEM((PAIRS,), jnp.int32),
        ],
    )
    def k(scores_hbm, idx_hbm, lens_hbm, x_hbm, xidx_hbm,
          row_v, idx_v, xbuf_v, xidx_v, lens_v):
        wid = lax.axis_index("s") * NC + lax.axis_index("c")
        b = wid // (NW // B)
        pltpu.sync_copy(scores_hbm.at[b], row_v)
        pltpu.sync_copy(lens_hbm, lens_v)
        lane = lax.broadcasted_iota(jnp.int32, (L,), 0)
        neg = jnp.full((L,), -1e9, jnp.float32)
        for jj in range(PPW):
            p = wid * PPW + jj
            pltpu.sync_copy(idx_hbm.at[p], idx_v)
            ln = plsc.load_gather(lens_v, [jnp.full((L,), p, jnp.int32)])

            def body(i, carry, ln=ln):
                best_val, best_idx = carry
                idxv = idx_v[pl.ds(i * L, L)]
                vals = plsc.load_gather(row_v, [idxv])
                xbuf_v[pl.ds(i * L, L)] = vals
                kv = lane + i * L
                mval = jnp.where(kv < ln, vals, neg)
                upd = mval > best_val
                return (jnp.where(upd, mval, best_val),
                        jnp.where(upd, kv, best_idx))

            bv0 = jnp.full((L,), -jnp.inf, jnp.float32)
            bi0 = jnp.zeros((L,), jnp.int32)
            bv, bi = lax.fori_loop(0, K // L, body, (bv0, bi0))
            mx = jnp.max(bv, axis=0)
            cand = jnp.where(bv == mx, bi, jnp.int32(K))
            amin = jnp.min(cand, axis=0)
            xidx_v[...] = jnp.full((L,), amin, jnp.int32)
            pltpu.sync_copy(xbuf_v, x_hbm.at[p])
            pltpu.sync_copy(xidx_v, xidx_hbm.at[p])

    return k(scores, idx, lens)


def kernel(stims, embed, atn_idx, lens):
    scores = _tc_scores(stims, embed)
    idx = atn_idx.reshape(PAIRS, K).astype(jnp.int32)
    lens_flat = lens.reshape(PAIRS).astype(jnp.int32)
    x_flat, xidx = _sc_gather_argmax(scores, idx, lens_flat)
    x = x_flat.reshape(B, J, K)
    xIdx = xidx[:, 0].reshape(B, J)
    return (x, xIdx)


# speedup vs baseline: 1.8089x; 1.0046x over previous
import functools

import jax
import jax.numpy as jnp
from jax import lax
from jax.experimental import pallas as pl
from jax.experimental.pallas import tpu as pltpu
from jax.experimental.pallas import tpu_sc as plsc

B, J, K, H, V = 16, 4, 2048, 256, 65536
PAIRS = B * J
L = 16
NC, NS = 2, 16
NW = NC * NS
PPW = PAIRS // NW
VBLK = 8192


def _tc_scores(stims, embed):
    def mm(stims_ref, emb_ref, out_ref):
        out_ref[...] = lax.dot_general(
            stims_ref[...], emb_ref[...],
            dimension_numbers=(((1,), (1,)), ((), ())),
            preferred_element_type=jnp.float32,
            precision=lax.Precision.HIGHEST,
        )

    return pl.pallas_call(
        mm,
        grid=(V // VBLK,),
        in_specs=[
            pl.BlockSpec((B, H), lambda i: (0, 0)),
            pl.BlockSpec((VBLK, H), lambda i: (i, 0)),
        ],
        out_specs=pl.BlockSpec((B, VBLK), lambda i: (0, i)),
        out_shape=jax.ShapeDtypeStruct((B, V), jnp.float32),
    )(stims, embed)


def _sc_gather_argmax(scores, idx, lens):
    mesh = plsc.VectorSubcoreMesh(core_axis_name="c", subcore_axis_name="s")

    @functools.partial(
        pl.kernel,
        mesh=mesh,
        compiler_params=pltpu.CompilerParams(needs_layout_passes=False),
        out_type=[
            jax.ShapeDtypeStruct((PAIRS, K), jnp.float32),
            jax.ShapeDtypeStruct((PAIRS, L), jnp.int32),
        ],
        scratch_types=[
            pltpu.VMEM((V,), jnp.float32),
            pltpu.VMEM((K,), jnp.int32),
            pltpu.VMEM((K,), jnp.float32),
            pltpu.VMEM((L,), jnp.int32),
            pltpu.VMEM((PAIRS,), jnp.int32),
        ],
    )
    def k(scores_hbm, idx_hbm, lens_hbm, x_hbm, xidx_hbm,
          row_v, idx_v, xbuf_v, xidx_v, lens_v):
        wid = lax.axis_index("s") * NC + lax.axis_index("c")
        b = wid // (NW // B)
        pltpu.sync_copy(scores_hbm.at[b], row_v)
        pltpu.sync_copy(lens_hbm, lens_v)
        lane = lax.broadcasted_iota(jnp.int32, (L,), 0)
        neg = jnp.full((L,), -1e9, jnp.float32)
        for jj in range(PPW):
            p = wid * PPW + jj
            pltpu.sync_copy(idx_hbm.at[p], idx_v)
            ln = plsc.load_gather(lens_v, [jnp.full((L,), p, jnp.int32)])

            def body(i, carry, ln=ln):
                best_val, best_idx = carry
                idxv = idx_v[pl.ds(i * L, L)]
                vals = plsc.load_gather(row_v, [idxv])
                xbuf_v[pl.ds(i * L, L)] = vals
                kv = lane + i * L
                mval = jnp.where(kv < ln, vals, neg)
                upd = mval > best_val
                return (jnp.where(upd, mval, best_val),
                        jnp.where(upd, kv, best_idx))

            bv0 = jnp.full((L,), -jnp.inf, jnp.float32)
            bi0 = jnp.zeros((L,), jnp.int32)
            bv, bi = lax.fori_loop(0, K // L, body, (bv0, bi0))
            mx = jnp.max(bv, axis=0)
            cand = jnp.where(bv == mx, bi, jnp.int32(K))
            amin = jnp.min(cand, axis=0)
            xidx_v[...] = jnp.full((L,), amin, jnp.int32)
            pltpu.sync_copy(xbuf_v, x_hbm.at[p])
            pltpu.sync_copy(xidx_v, xidx_hbm.at[p])

    return k(scores, idx, lens)


def kernel(stims, embed, atn_idx, lens):
    scores = _tc_scores(stims, embed)
    idx = atn_idx.reshape(PAIRS, K).astype(jnp.int32)
    lens_flat = lens.reshape(PAIRS).astype(jnp.int32)
    x_flat, xidx = _sc_gather_argmax(scores, idx, lens_flat)
    x = x_flat.reshape(B, J, K)
    xIdx = xidx[:, 0].reshape(B, J)
    return (x, xIdx)
